# baseline (speedup 1.0000x reference)
import jax
import jax.numpy as jnp
from jax.experimental import pallas as pl

B = 16384
D = 128
BLK = 1024


def _row_dot_kernel(gu_ref, gi_ref, bu_ref, bi_ref, mu_ref, out_ref):
    p = gu_ref[...] * gi_ref[...]
    ones = jnp.ones((1, D), dtype=jnp.float32)
    s = jax.lax.dot_general(
        ones, p, (((1,), (1,)), ((), ())),
        preferred_element_type=jnp.float32,
        precision=jax.lax.Precision.HIGHEST,
    )
    out_ref[...] = s.reshape(BLK) + bu_ref[...] + bi_ref[...] + mu_ref[0, 0]


def kernel(gu, gi, bu, bi, Mu):
    bu_f = bu.reshape(B)
    bi_f = bi.reshape(B)
    grid = (B // BLK,)
    out = pl.pallas_call(
        _row_dot_kernel,
        grid=grid,
        in_specs=[
            pl.BlockSpec((BLK, D), lambda i: (i, 0)),
            pl.BlockSpec((BLK, D), lambda i: (i, 0)),
            pl.BlockSpec((BLK,), lambda i: (i,)),
            pl.BlockSpec((BLK,), lambda i: (i,)),
            pl.BlockSpec((1, 1), lambda i: (0, 0)),
        ],
        out_specs=pl.BlockSpec((BLK,), lambda i: (i,)),
        out_shape=jax.ShapeDtypeStruct((B,), jnp.float32),
    )(gu, gi, bu_f, bi_f, Mu)
    return out


# speedup vs baseline: 1.7378x; 1.7378x over previous
import jax
import jax.numpy as jnp
from jax.experimental import pallas as pl

B = 16384
D = 128
BLK = 2048


def _row_dot_kernel(gu_ref, gi_ref, bu_ref, bi_ref, mu_ref, out_ref):
    p = gu_ref[...] * gi_ref[...]
    ones = jnp.ones((1, D), dtype=jnp.float32)
    s = jax.lax.dot_general(
        ones, p, (((1,), (1,)), ((), ())),
        preferred_element_type=jnp.float32,
    )
    out_ref[...] = s.reshape(BLK) + bu_ref[...] + bi_ref[...] + mu_ref[0, 0]


def kernel(gu, gi, bu, bi, Mu):
    bu_f = bu.reshape(B)
    bi_f = bi.reshape(B)
    grid = (B // BLK,)
    out = pl.pallas_call(
        _row_dot_kernel,
        grid=grid,
        in_specs=[
            pl.BlockSpec((BLK, D), lambda i: (i, 0)),
            pl.BlockSpec((BLK, D), lambda i: (i, 0)),
            pl.BlockSpec((BLK,), lambda i: (i,)),
            pl.BlockSpec((BLK,), lambda i: (i,)),
            pl.BlockSpec((1, 1), lambda i: (0, 0)),
        ],
        out_specs=pl.BlockSpec((BLK,), lambda i: (i,)),
        out_shape=jax.ShapeDtypeStruct((B,), jnp.float32),
    )(gu, gi, bu_f, bi_f, Mu)
    return out


# speedup vs baseline: 2.1162x; 1.2177x over previous
import jax
import jax.numpy as jnp
from jax.experimental import pallas as pl

B = 16384
D = 128
BLK = 4096


def _row_dot_kernel(gu_ref, gi_ref, bu_ref, bi_ref, mu_ref, out_ref):
    p = gu_ref[...] * gi_ref[...]
    ones = jnp.ones((1, D), dtype=jnp.float32)
    s = jax.lax.dot_general(
        ones, p, (((1,), (1,)), ((), ())),
        preferred_element_type=jnp.float32,
    )
    out_ref[...] = s.reshape(BLK) + bu_ref[...] + bi_ref[...] + mu_ref[0, 0]


def kernel(gu, gi, bu, bi, Mu):
    bu_f = bu.reshape(B)
    bi_f = bi.reshape(B)
    grid = (B // BLK,)
    out = pl.pallas_call(
        _row_dot_kernel,
        grid=grid,
        in_specs=[
            pl.BlockSpec((BLK, D), lambda i: (i, 0)),
            pl.BlockSpec((BLK, D), lambda i: (i, 0)),
            pl.BlockSpec((BLK,), lambda i: (i,)),
            pl.BlockSpec((BLK,), lambda i: (i,)),
            pl.BlockSpec((1, 1), lambda i: (0, 0)),
        ],
        out_specs=pl.BlockSpec((BLK,), lambda i: (i,)),
        out_shape=jax.ShapeDtypeStruct((B,), jnp.float32),
    )(gu, gi, bu_f, bi_f, Mu)
    return out


# speedup vs baseline: 2.1571x; 1.0194x over previous
import jax
import jax.numpy as jnp
from jax.experimental import pallas as pl

B = 16384
D = 128
BLK = 8192


def _row_dot_kernel(gu_ref, gi_ref, bu_ref, bi_ref, mu_ref, out_ref):
    p = gu_ref[...] * gi_ref[...]
    ones = jnp.ones((1, D), dtype=jnp.float32)
    s = jax.lax.dot_general(
        ones, p, (((1,), (1,)), ((), ())),
        preferred_element_type=jnp.float32,
    )
    out_ref[...] = s.reshape(BLK) + bu_ref[...] + bi_ref[...] + mu_ref[0, 0]


def kernel(gu, gi, bu, bi, Mu):
    bu_f = bu.reshape(B)
    bi_f = bi.reshape(B)
    grid = (B // BLK,)
    out = pl.pallas_call(
        _row_dot_kernel,
        grid=grid,
        in_specs=[
            pl.BlockSpec((BLK, D), lambda i: (i, 0)),
            pl.BlockSpec((BLK, D), lambda i: (i, 0)),
            pl.BlockSpec((BLK,), lambda i: (i,)),
            pl.BlockSpec((BLK,), lambda i: (i,)),
            pl.BlockSpec((1, 1), lambda i: (0, 0)),
        ],
        out_specs=pl.BlockSpec((BLK,), lambda i: (i,)),
        out_shape=jax.ShapeDtypeStruct((B,), jnp.float32),
    )(gu, gi, bu_f, bi_f, Mu)
    return out
